# baseline (speedup 1.0000x reference)
import functools

import jax
import jax.numpy as jnp
import numpy as np
from jax import lax
from jax.experimental import pallas as pl
from jax.experimental.pallas import tpu as pltpu
from jax.experimental.pallas import tpu_sc as plsc

N = 10000
E = 160000
D_IN = 128
D_EDGE = 4
HID = 50
NB = 10
FC_H = 100
MAX_R = 3.5
DIMS = [(D_IN, HID), (HID, HID), (HID, HID), (HID, D_IN)]
WIDTHS = [128, 128, 128, 128]

NC = 2
NS = 16
NW = NC * NS
K = 128
C = 40
EW = C * K
EP = NW * EW
NP = 10240
SHARE = NP // NS

_mesh = plsc.VectorSubcoreMesh(core_axis_name="c", subcore_axis_name="s")
_sc_params = pltpu.CompilerParams(needs_layout_passes=False,
                                 use_tc_tiling_on_sc=True)


FCH = 4
EWC = EW // FCH


@functools.partial(
    pl.kernel,
    out_type=jax.ShapeDtypeStruct((14, EP), jnp.float32),
    mesh=_mesh,
    compiler_params=_sc_params,
    scratch_types=[
        pltpu.VMEM((N * 3,), jnp.float32),
        pltpu.VMEM((C, K), jnp.int32),
        pltpu.VMEM((C, K), jnp.int32),
        pltpu.VMEM((EW * 4,), jnp.float32),
        pltpu.VMEM((14, EWC), jnp.float32),
    ],
)
def _emb_sc(pos_hbm, src_hbm, dst_hbm, ea_hbm, out_hbm,
            pos_v, src_v, dst_v, ea_v, f_v):
    cid = lax.axis_index("c")
    sid = lax.axis_index("s")
    wid = sid * NC + cid
    base = wid * EW
    pltpu.sync_copy(pos_hbm, pos_v)
    pltpu.sync_copy(src_hbm.at[pl.ds(wid * C, C)], src_v)
    pltpu.sync_copy(dst_hbm.at[pl.ds(wid * C, C)], dst_v)
    pltpu.sync_copy(ea_hbm.at[pl.ds(base * 4, EW * 4)], ea_v)

    rstep = jnp.float32(1.0 / _STEP)

    def chunk(cc, carry):
        def body(i, carry2):
            g = cc * (EWC // 16) + i
            row = g // 8
            col = (g % 8) * 16
            sl = pl.ds(col, 16)
            osl = pl.ds(i * 16, 16)
            s3 = src_v[row, sl] * 3
            d3 = dst_v[row, sl] * 3
            dx = plsc.load_gather(pos_v, [s3]) - plsc.load_gather(pos_v, [d3])
            dy = plsc.load_gather(pos_v, [s3 + 1]) - plsc.load_gather(pos_v, [d3 + 1])
            dz = plsc.load_gather(pos_v, [s3 + 2]) - plsc.load_gather(pos_v, [d3 + 2])
            u = dx * dx + dy * dy + dz * dz + 1e-12
            yb = plsc.bitcast(
                lax.shift_right_logical(plsc.bitcast(u, jnp.int32), 1)
                + jnp.int32(0x1FBD1DF5), jnp.float32)
            yb = 0.5 * (yb + u / yb)
            yb = 0.5 * (yb + u / yb)
            ln = 0.5 * (yb + u / yb)
            e16 = (cc * EWC + i * 16)
            for k in range(NB):
                diff = ln * rstep - jnp.float32(k + 1)
                xp = diff + 1.0
                xm = 1.0 - diff
                sp = jnp.where(xp > 0.0,
                               jnp.exp(-1.0 / jnp.where(xp > 0.0, xp, 1.0)), 0.0)
                sm = jnp.where(xm > 0.0,
                               jnp.exp(-1.0 / jnp.where(xm > 0.0, xm, 1.0)), 0.0)
                f_v[k, osl] = jnp.float32(_C0) * sp * sm
            ebase = e16 * 4
            for k in range(4):
                f_v[10 + k, osl] = plsc.load_gather(
                    ea_v, [lax.iota(jnp.int32, 16) * 4 + (ebase + k)])
            return carry2

        lax.fori_loop(0, EWC // 16, body, 0, unroll=2)
        pltpu.sync_copy(f_v, out_hbm.at[:, pl.ds(base + cc * EWC, EWC)])
        return carry

    lax.fori_loop(0, FCH, chunk, 0)


def _make_scatter_sc(W):
    @functools.partial(
        pl.kernel,
        out_type=jax.ShapeDtypeStruct((NC, NP, W), jnp.float32),
        mesh=_mesh,
        compiler_params=_sc_params,
        scratch_types=[
            pltpu.VMEM((C, K), jnp.int32),
            pltpu.VMEM((C, K), jnp.int32),
            pltpu.VMEM((EW,), jnp.float32),
            pltpu.VMEM((K, W), jnp.float32),
            pltpu.VMEM((K, W), jnp.float32),
            pltpu.VMEM_SHARED((NP, W), jnp.float32),
            pltpu.SemaphoreType.DMA,
            pltpu.SemaphoreType.DMA,
        ],
    )
    def sck(h1_hbm, gate_hbm, srcm_hbm, dstm_hbm, out_hbm,
            src_v, dst_v, gate_v, rows_a, rows_b, agg_sh, sem_a, sem_b):
        cid = lax.axis_index("c")
        sid = lax.axis_index("s")
        wid = sid * NC + cid
        pltpu.sync_copy(srcm_hbm.at[pl.ds(wid * C, C)], src_v)
        pltpu.sync_copy(dstm_hbm.at[pl.ds(wid * C, C)], dst_v)
        pltpu.sync_copy(gate_hbm.at[pl.ds(wid * EW, EW)], gate_v)

        zv = jnp.zeros((16,), jnp.float32)

        def zbody(r, carry):
            for v in range(W // 16):
                rows_a[r, pl.ds(v * 16, 16)] = zv
            return carry

        lax.fori_loop(0, K, zbody, 0, unroll=8)
        rbase = sid * SHARE
        nfull = SHARE // K
        for t in range(nfull):
            pltpu.sync_copy(rows_a, agg_sh.at[pl.ds(rbase + t * K, K)])
        plsc.subcore_barrier()

        def scale(rows, gbase):
            def ebody(e, carry):
                gv = plsc.load_gather(gate_v, [jnp.full((16,), gbase + e, jnp.int32)])
                for v in range(W // 16):
                    sl = pl.ds(v * 16, 16)
                    rows[e, sl] = rows[e, sl] * gv
                return carry

            lax.fori_loop(0, K, ebody, 0, unroll=4)

        pltpu.async_copy(h1_hbm.at[src_v.at[0]], rows_a, sem_a)
        pltpu.async_copy(h1_hbm.at[src_v.at[1]], rows_b, sem_b)

        def jbody(i, carry):
            ja = 2 * i
            jb = 2 * i + 1
            pltpu.make_async_copy(h1_hbm.at[src_v.at[ja]], rows_a, sem_a).wait()
            scale(rows_a, ja * K)
            pltpu.sync_copy(rows_a, agg_sh.at[dst_v.at[ja]], add=True)

            @pl.when(ja + 2 < C)
            def _():
                pltpu.async_copy(h1_hbm.at[src_v.at[ja + 2]], rows_a, sem_a)

            pltpu.make_async_copy(h1_hbm.at[src_v.at[jb]], rows_b, sem_b).wait()
            scale(rows_b, jb * K)
            pltpu.sync_copy(rows_b, agg_sh.at[dst_v.at[jb]], add=True)

            @pl.when(jb + 2 < C)
            def _():
                pltpu.async_copy(h1_hbm.at[src_v.at[jb + 2]], rows_b, sem_b)

            return carry

        lax.fori_loop(0, C // 2, jbody, 0)
        plsc.subcore_barrier()
        pltpu.sync_copy(agg_sh.at[pl.ds(rbase, SHARE)],
                        out_hbm.at[cid, pl.ds(rbase, SHARE)])

    return sck


_scatter_sc = {W: _make_scatter_sc(W) for W in (128,)}


_EB = 2048

_STEP = float(MAX_R / (NB + 1))
_C0 = float(1.14136 * np.exp(2.0) * np.sqrt(NB))


def _sus(t):
    ts = jnp.where(t > 0.0, t, 1.0)
    return jnp.where(t > 0.0, jnp.exp(-1.0 / ts), 0.0)


def _gates_body(f_ref, w1t_ref, b1_ref, w2t_ref, out_ref):
    emb = f_ref[0:NB, :]
    t = jnp.dot(w1t_ref[...], emb, preferred_element_type=jnp.float32)
    t = t + b1_ref[...]
    t = t / (1.0 + jnp.exp(-t))
    w = jnp.dot(w2t_ref[...], t, preferred_element_type=jnp.float32)
    acc = f_ref[10, :] * w[0, :]
    for k in range(1, 4):
        acc = acc + f_ref[10 + k, :] * w[k, :]
    out_ref[...] = acc[None, :]


_EB = 4096


def _gates_tc(f, w1t, b1, w2t):
    grid = EP // _EB
    return pl.pallas_call(
        _gates_body,
        grid=(grid,),
        in_specs=[
            pl.BlockSpec((14, _EB), lambda i: (0, i)),
            pl.BlockSpec((FC_H, NB), lambda i: (0, 0)),
            pl.BlockSpec((FC_H, 1), lambda i: (0, 0)),
            pl.BlockSpec((D_EDGE, FC_H), lambda i: (0, 0)),
        ],
        out_specs=pl.BlockSpec((1, _EB), lambda i: (0, i)),
        out_shape=jax.ShapeDtypeStruct((1, EP), jnp.float32),
    )(f, w1t, b1, w2t)


_NB_ROWS = 1000


def _pre_body(x_ref, na_ref, w1_ref, wsc_ref, h1_ref, sc_ref):
    xb = x_ref[...]
    h1_ref[...] = jnp.dot(xb, w1_ref[...], preferred_element_type=jnp.float32)
    sc_ref[...] = jnp.dot(xb * na_ref[...], wsc_ref[...],
                          preferred_element_type=jnp.float32)


def _pre_tc(x, na, w1p, wscp):
    grid = N // _NB_ROWS
    return pl.pallas_call(
        _pre_body,
        grid=(grid,),
        in_specs=[
            pl.BlockSpec((_NB_ROWS, 128), lambda i: (i, 0)),
            pl.BlockSpec((_NB_ROWS, 1), lambda i: (i, 0)),
            pl.BlockSpec((128, 128), lambda i: (0, 0)),
            pl.BlockSpec((128, 128), lambda i: (0, 0)),
        ],
        out_specs=[
            pl.BlockSpec((_NB_ROWS, 128), lambda i: (i, 0)),
            pl.BlockSpec((_NB_ROWS, 128), lambda i: (i, 0)),
        ],
        out_shape=[
            jax.ShapeDtypeStruct((N, 128), jnp.float32),
            jax.ShapeDtypeStruct((N, 128), jnp.float32),
        ],
    )(x, na, w1p, wscp)


def _mid_body(sc_ref, agg_ref, na_ref, w2_ref, w1n_ref, wscn_ref,
              h1_ref, scn_ref):
    a = (agg_ref[0] + agg_ref[1]) * 0.25
    h = sc_ref[...] + jnp.dot(a, w2_ref[...], preferred_element_type=jnp.float32)
    h = h * jax.nn.sigmoid(h)
    h1_ref[...] = jnp.dot(h, w1n_ref[...], preferred_element_type=jnp.float32)
    scn_ref[...] = jnp.dot(h * na_ref[...], wscn_ref[...],
                           preferred_element_type=jnp.float32)


def _mid_tc(sc, agg2, na, w2p, w1pn, wscpn, W, Wn):
    grid = N // _NB_ROWS
    return pl.pallas_call(
        _mid_body,
        grid=(grid,),
        in_specs=[
            pl.BlockSpec((_NB_ROWS, 128), lambda i: (i, 0)),
            pl.BlockSpec((2, _NB_ROWS, W), lambda i: (0, i, 0)),
            pl.BlockSpec((_NB_ROWS, 1), lambda i: (i, 0)),
            pl.BlockSpec((W, 128), lambda i: (0, 0)),
            pl.BlockSpec((128, Wn), lambda i: (0, 0)),
            pl.BlockSpec((128, 128), lambda i: (0, 0)),
        ],
        out_specs=[
            pl.BlockSpec((_NB_ROWS, Wn), lambda i: (i, 0)),
            pl.BlockSpec((_NB_ROWS, 128), lambda i: (i, 0)),
        ],
        out_shape=[
            jax.ShapeDtypeStruct((N, Wn), jnp.float32),
            jax.ShapeDtypeStruct((N, 128), jnp.float32),
        ],
    )(sc, agg2, na, w2p, w1pn, wscpn)


def _post_body(sc_ref, agg_ref, w2_ref, out_ref):
    a = (agg_ref[0] + agg_ref[1]) * 0.25
    out_ref[...] = sc_ref[...] + jnp.dot(a, w2_ref[...],
                                         preferred_element_type=jnp.float32)


def _post_tc(sc, agg2, w2p, W):
    grid = N // _NB_ROWS
    return pl.pallas_call(
        _post_body,
        grid=(grid,),
        in_specs=[
            pl.BlockSpec((_NB_ROWS, 128), lambda i: (i, 0)),
            pl.BlockSpec((2, _NB_ROWS, W), lambda i: (0, i, 0)),
            pl.BlockSpec((W, 128), lambda i: (0, 0)),
        ],
        out_specs=pl.BlockSpec((_NB_ROWS, 128), lambda i: (i, 0)),
        out_shape=jax.ShapeDtypeStruct((N, 128), jnp.float32),
    )(sc, agg2, w2p)


def _pad2(a, r, c):
    return jnp.pad(a, ((0, r - a.shape[0]), (0, c - a.shape[1])))


@jax.jit
def _run(x, node_attr, edge_attr, pos, edge_index, params):
    src = edge_index[0].astype(jnp.int32)
    dst = edge_index[1].astype(jnp.int32)
    npad = EP - E
    pad_idx = jnp.arange(npad, dtype=jnp.int32) % N
    src_p = jnp.concatenate([src, pad_idx])
    dst_p = jnp.concatenate([dst, pad_idx])
    ea_p = jnp.concatenate(
        [edge_attr, jnp.zeros((npad, D_EDGE), jnp.float32)])
    srcm = src_p.reshape(NW * C, K)
    dstm = dst_p.reshape(NW * C, K)

    fmat = _emb_sc(pos.reshape(-1), srcm, dstm, ea_p.reshape(-1))

    def gates_l(l):
        p = params[l]
        return _gates_tc(fmat, p["fcW1"].T, p["fcb1"][:, None],
                         p["fcW2"].T).reshape(EP)

    w1p = [_pad2(params[l]["Wlin1"], 128, WIDTHS[l]) for l in range(4)]
    wscp = [_pad2(params[l]["Wsc"], 128, 128) for l in range(4)]
    w2p = [_pad2(params[l]["Wlin2"], WIDTHS[l], 128) for l in range(4)]

    h1p, scp = _pre_tc(x, node_attr, w1p[0], wscp[0])
    gate_cur = gates_l(0)
    for l in range(4):
        Wl = WIDTHS[l]
        agg2 = _scatter_sc[Wl](h1p, gate_cur, srcm, dstm)
        if l < 3:
            gate_cur = gates_l(l + 1)
            h1p, scp = _mid_tc(scp, agg2, node_attr, w2p[l],
                               w1p[l + 1], wscp[l + 1], Wl, WIDTHS[l + 1])
        else:
            out = _post_tc(scp, agg2, w2p[l], Wl)
    return out


def kernel(x, node_attr, edge_attr, pos, edge_index, params):
    return _run(x, node_attr, edge_attr, pos, edge_index, params)


# speedup vs baseline: 1.3300x; 1.3300x over previous
import functools

import jax
import jax.numpy as jnp
import numpy as np
from jax import lax
from jax.experimental import pallas as pl
from jax.experimental.pallas import tpu as pltpu
from jax.experimental.pallas import tpu_sc as plsc

N = 10000
E = 160000
D_IN = 128
D_EDGE = 4
HID = 50
NB = 10
FC_H = 100
MAX_R = 3.5
DIMS = [(D_IN, HID), (HID, HID), (HID, HID), (HID, D_IN)]
WIDTHS = [128, 128, 128, 128]

NC = 2
NS = 16
NW = NC * NS
K = 128
C = 40
EW = C * K
EP = NW * EW
NP = 10240
SHARE = NP // NS

_mesh = plsc.VectorSubcoreMesh(core_axis_name="c", subcore_axis_name="s")
_sc_params = pltpu.CompilerParams(needs_layout_passes=False,
                                 use_tc_tiling_on_sc=True)


FCH = 4
EWC = EW // FCH


@functools.partial(
    pl.kernel,
    out_type=jax.ShapeDtypeStruct((NB, EP), jnp.float32),
    mesh=_mesh,
    compiler_params=_sc_params,
    scratch_types=[
        pltpu.VMEM((N * 3,), jnp.float32),
        pltpu.VMEM((C, K), jnp.int32),
        pltpu.VMEM((C, K), jnp.int32),
        pltpu.VMEM((NB, EWC), jnp.float32),
    ],
)
def _emb_sc(pos_hbm, src_hbm, dst_hbm, out_hbm,
            pos_v, src_v, dst_v, f_v):
    cid = lax.axis_index("c")
    sid = lax.axis_index("s")
    wid = sid * NC + cid
    base = wid * EW
    pltpu.sync_copy(pos_hbm, pos_v)
    pltpu.sync_copy(src_hbm.at[pl.ds(wid * C, C)], src_v)
    pltpu.sync_copy(dst_hbm.at[pl.ds(wid * C, C)], dst_v)

    rstep = jnp.float32(1.0 / _STEP)

    def chunk(cc, carry):
        def body(i, carry2):
            g = cc * (EWC // 16) + i
            row = g // 8
            col = (g % 8) * 16
            sl = pl.ds(col, 16)
            osl = pl.ds(i * 16, 16)
            s3 = src_v[row, sl] * 3
            d3 = dst_v[row, sl] * 3
            dx = plsc.load_gather(pos_v, [s3]) - plsc.load_gather(pos_v, [d3])
            dy = plsc.load_gather(pos_v, [s3 + 1]) - plsc.load_gather(pos_v, [d3 + 1])
            dz = plsc.load_gather(pos_v, [s3 + 2]) - plsc.load_gather(pos_v, [d3 + 2])
            u = dx * dx + dy * dy + dz * dz + 1e-12
            yb = plsc.bitcast(
                lax.shift_right_logical(plsc.bitcast(u, jnp.int32), 1)
                + jnp.int32(0x1FBD1DF5), jnp.float32)
            yb = 0.5 * (yb + u / yb)
            yb = 0.5 * (yb + u / yb)
            ln = 0.5 * (yb + u / yb)
            lr = ln * rstep
            for k in range(NB):
                diff = lr - jnp.float32(k + 1)
                q = 1.0 - diff * diff
                inr = q > 0.0
                e = jnp.exp(-2.0 / jnp.where(inr, q, 1.0))
                f_v[k, osl] = jnp.where(inr, jnp.float32(_C0) * e, 0.0)
            return carry2

        lax.fori_loop(0, EWC // 16, body, 0, unroll=2)
        pltpu.sync_copy(f_v, out_hbm.at[:, pl.ds(base + cc * EWC, EWC)])
        return carry

    lax.fori_loop(0, FCH, chunk, 0)


def _make_scatter_sc(W):
    @functools.partial(
        pl.kernel,
        out_type=jax.ShapeDtypeStruct((NC, NP, W), jnp.float32),
        mesh=_mesh,
        compiler_params=_sc_params,
        scratch_types=[
            pltpu.VMEM((C, K), jnp.int32),
            pltpu.VMEM((C, K), jnp.int32),
            pltpu.VMEM((EW,), jnp.float32),
            pltpu.VMEM((K, W), jnp.float32),
            pltpu.VMEM((K, W), jnp.float32),
            pltpu.VMEM_SHARED((NP, W), jnp.float32),
            pltpu.SemaphoreType.DMA,
            pltpu.SemaphoreType.DMA,
        ],
    )
    def sck(h1_hbm, gate_hbm, srcm_hbm, dstm_hbm, out_hbm,
            src_v, dst_v, gate_v, rows_a, rows_b, agg_sh, sem_a, sem_b):
        cid = lax.axis_index("c")
        sid = lax.axis_index("s")
        wid = sid * NC + cid
        pltpu.sync_copy(srcm_hbm.at[pl.ds(wid * C, C)], src_v)
        pltpu.sync_copy(dstm_hbm.at[pl.ds(wid * C, C)], dst_v)
        pltpu.sync_copy(gate_hbm.at[pl.ds(wid * EW, EW)], gate_v)

        zv = jnp.zeros((16,), jnp.float32)

        def zbody(r, carry):
            for v in range(W // 16):
                rows_a[r, pl.ds(v * 16, 16)] = zv
            return carry

        lax.fori_loop(0, K, zbody, 0, unroll=8)
        rbase = sid * SHARE
        nfull = SHARE // K
        for t in range(nfull):
            pltpu.sync_copy(rows_a, agg_sh.at[pl.ds(rbase + t * K, K)])
        plsc.subcore_barrier()

        def scale(rows, gbase):
            def ebody(e, carry):
                gv = plsc.load_gather(gate_v, [jnp.full((16,), gbase + e, jnp.int32)])
                for v in range(W // 16):
                    sl = pl.ds(v * 16, 16)
                    rows[e, sl] = rows[e, sl] * gv
                return carry

            lax.fori_loop(0, K, ebody, 0, unroll=4)

        pltpu.async_copy(h1_hbm.at[src_v.at[0]], rows_a, sem_a)
        pltpu.async_copy(h1_hbm.at[src_v.at[1]], rows_b, sem_b)

        def jbody(i, carry):
            ja = 2 * i
            jb = 2 * i + 1
            pltpu.make_async_copy(h1_hbm.at[src_v.at[ja]], rows_a, sem_a).wait()
            scale(rows_a, ja * K)
            pltpu.sync_copy(rows_a, agg_sh.at[dst_v.at[ja]], add=True)

            @pl.when(ja + 2 < C)
            def _():
                pltpu.async_copy(h1_hbm.at[src_v.at[ja + 2]], rows_a, sem_a)

            pltpu.make_async_copy(h1_hbm.at[src_v.at[jb]], rows_b, sem_b).wait()
            scale(rows_b, jb * K)
            pltpu.sync_copy(rows_b, agg_sh.at[dst_v.at[jb]], add=True)

            @pl.when(jb + 2 < C)
            def _():
                pltpu.async_copy(h1_hbm.at[src_v.at[jb + 2]], rows_b, sem_b)

            return carry

        lax.fori_loop(0, C // 2, jbody, 0)
        plsc.subcore_barrier()
        pltpu.sync_copy(agg_sh.at[pl.ds(rbase, SHARE)],
                        out_hbm.at[cid, pl.ds(rbase, SHARE)])

    return sck


_scatter_sc = {W: _make_scatter_sc(W) for W in (128,)}


_EB = 2048

_STEP = float(MAX_R / (NB + 1))
_C0 = float(1.14136 * np.exp(2.0) * np.sqrt(NB))


def _sus(t):
    ts = jnp.where(t > 0.0, t, 1.0)
    return jnp.where(t > 0.0, jnp.exp(-1.0 / ts), 0.0)


def _gates_body(f_ref, ea_ref, w1t_ref, b1_ref, w2t_ref, out_ref):
    t = jnp.dot(w1t_ref[...], f_ref[...], preferred_element_type=jnp.float32)
    t = t + b1_ref[...]
    t = t / (1.0 + jnp.exp(-t))
    w = jnp.dot(w2t_ref[...], t, preferred_element_type=jnp.float32)
    acc = ea_ref[0, :] * w[0, :]
    for k in range(1, 4):
        acc = acc + ea_ref[k, :] * w[k, :]
    out_ref[...] = acc[None, :]


_EB = 4096


def _gates_tc(f, ea_t, w1t, b1, w2t):
    grid = EP // _EB
    return pl.pallas_call(
        _gates_body,
        grid=(grid,),
        in_specs=[
            pl.BlockSpec((NB, _EB), lambda i: (0, i)),
            pl.BlockSpec((D_EDGE, _EB), lambda i: (0, i)),
            pl.BlockSpec((FC_H, NB), lambda i: (0, 0)),
            pl.BlockSpec((FC_H, 1), lambda i: (0, 0)),
            pl.BlockSpec((D_EDGE, FC_H), lambda i: (0, 0)),
        ],
        out_specs=pl.BlockSpec((1, _EB), lambda i: (0, i)),
        out_shape=jax.ShapeDtypeStruct((1, EP), jnp.float32),
    )(f, ea_t, w1t, b1, w2t)


_NB_ROWS = 1000


def _pre_body(x_ref, na_ref, w1_ref, wsc_ref, h1_ref, sc_ref):
    xb = x_ref[...]
    h1_ref[...] = jnp.dot(xb, w1_ref[...], preferred_element_type=jnp.float32)
    sc_ref[...] = jnp.dot(xb * na_ref[...], wsc_ref[...],
                          preferred_element_type=jnp.float32)


def _pre_tc(x, na, w1p, wscp):
    grid = N // _NB_ROWS
    return pl.pallas_call(
        _pre_body,
        grid=(grid,),
        in_specs=[
            pl.BlockSpec((_NB_ROWS, 128), lambda i: (i, 0)),
            pl.BlockSpec((_NB_ROWS, 1), lambda i: (i, 0)),
            pl.BlockSpec((128, 128), lambda i: (0, 0)),
            pl.BlockSpec((128, 128), lambda i: (0, 0)),
        ],
        out_specs=[
            pl.BlockSpec((_NB_ROWS, 128), lambda i: (i, 0)),
            pl.BlockSpec((_NB_ROWS, 128), lambda i: (i, 0)),
        ],
        out_shape=[
            jax.ShapeDtypeStruct((N, 128), jnp.float32),
            jax.ShapeDtypeStruct((N, 128), jnp.float32),
        ],
    )(x, na, w1p, wscp)


def _mid_body(sc_ref, agg_ref, na_ref, w2_ref, w1n_ref, wscn_ref,
              h1_ref, scn_ref):
    a = (agg_ref[0] + agg_ref[1]) * 0.25
    h = sc_ref[...] + jnp.dot(a, w2_ref[...], preferred_element_type=jnp.float32)
    h = h * jax.nn.sigmoid(h)
    h1_ref[...] = jnp.dot(h, w1n_ref[...], preferred_element_type=jnp.float32)
    scn_ref[...] = jnp.dot(h * na_ref[...], wscn_ref[...],
                           preferred_element_type=jnp.float32)


def _mid_tc(sc, agg2, na, w2p, w1pn, wscpn, W, Wn):
    grid = N // _NB_ROWS
    return pl.pallas_call(
        _mid_body,
        grid=(grid,),
        in_specs=[
            pl.BlockSpec((_NB_ROWS, 128), lambda i: (i, 0)),
            pl.BlockSpec((2, _NB_ROWS, W), lambda i: (0, i, 0)),
            pl.BlockSpec((_NB_ROWS, 1), lambda i: (i, 0)),
            pl.BlockSpec((W, 128), lambda i: (0, 0)),
            pl.BlockSpec((128, Wn), lambda i: (0, 0)),
            pl.BlockSpec((128, 128), lambda i: (0, 0)),
        ],
        out_specs=[
            pl.BlockSpec((_NB_ROWS, Wn), lambda i: (i, 0)),
            pl.BlockSpec((_NB_ROWS, 128), lambda i: (i, 0)),
        ],
        out_shape=[
            jax.ShapeDtypeStruct((N, Wn), jnp.float32),
            jax.ShapeDtypeStruct((N, 128), jnp.float32),
        ],
    )(sc, agg2, na, w2p, w1pn, wscpn)


def _post_body(sc_ref, agg_ref, w2_ref, out_ref):
    a = (agg_ref[0] + agg_ref[1]) * 0.25
    out_ref[...] = sc_ref[...] + jnp.dot(a, w2_ref[...],
                                         preferred_element_type=jnp.float32)


def _post_tc(sc, agg2, w2p, W):
    grid = N // _NB_ROWS
    return pl.pallas_call(
        _post_body,
        grid=(grid,),
        in_specs=[
            pl.BlockSpec((_NB_ROWS, 128), lambda i: (i, 0)),
            pl.BlockSpec((2, _NB_ROWS, W), lambda i: (0, i, 0)),
            pl.BlockSpec((W, 128), lambda i: (0, 0)),
        ],
        out_specs=pl.BlockSpec((_NB_ROWS, 128), lambda i: (i, 0)),
        out_shape=jax.ShapeDtypeStruct((N, 128), jnp.float32),
    )(sc, agg2, w2p)


def _pad2(a, r, c):
    return jnp.pad(a, ((0, r - a.shape[0]), (0, c - a.shape[1])))


@jax.jit
def _run(x, node_attr, edge_attr, pos, edge_index, params):
    src = edge_index[0].astype(jnp.int32)
    dst = edge_index[1].astype(jnp.int32)
    npad = EP - E
    pad_idx = jnp.arange(npad, dtype=jnp.int32) % N
    src_p = jnp.concatenate([src, pad_idx])
    dst_p = jnp.concatenate([dst, pad_idx])
    ea_t = jnp.pad(edge_attr.T, ((0, 0), (0, npad)))
    srcm = src_p.reshape(NW * C, K)
    dstm = dst_p.reshape(NW * C, K)

    fmat = _emb_sc(pos.reshape(-1), srcm, dstm)

    def gates_l(l):
        p = params[l]
        return _gates_tc(fmat, ea_t, p["fcW1"].T, p["fcb1"][:, None],
                         p["fcW2"].T).reshape(EP)

    w1p = [_pad2(params[l]["Wlin1"], 128, WIDTHS[l]) for l in range(4)]
    wscp = [_pad2(params[l]["Wsc"], 128, 128) for l in range(4)]
    w2p = [_pad2(params[l]["Wlin2"], WIDTHS[l], 128) for l in range(4)]

    h1p, scp = _pre_tc(x, node_attr, w1p[0], wscp[0])
    gate_cur = gates_l(0)
    for l in range(4):
        Wl = WIDTHS[l]
        agg2 = _scatter_sc[Wl](h1p, gate_cur, srcm, dstm)
        if l < 3:
            gate_cur = gates_l(l + 1)
            h1p, scp = _mid_tc(scp, agg2, node_attr, w2p[l],
                               w1p[l + 1], wscp[l + 1], Wl, WIDTHS[l + 1])
        else:
            out = _post_tc(scp, agg2, w2p[l], Wl)
    return out


def kernel(x, node_attr, edge_attr, pos, edge_index, params):
    return _run(x, node_attr, edge_attr, pos, edge_index, params)
